# baseline (speedup 1.0000x reference)
import jax
import jax.numpy as jnp
from jax.experimental import pallas as pl
from jax.experimental.pallas import tpu as pltpu

NE = 1024
ED = 64
NPIX = 16384
NB = 16


def _vq_body(z_ref, w_ref, idx_ref, zq_ref, cnt_ref, stats_ref):
    t = pl.program_id(0)
    zb = z_ref[0]
    zt = zb.T
    w = w_ref[...]
    m = jax.lax.dot_general(zt, w, (((1,), (1,)), ((), ())),
                            preferred_element_type=jnp.float32)
    zsq = jnp.sum(zt * zt, axis=1, keepdims=True)
    wsq = jnp.sum(w * w, axis=1)[None, :]
    dist = (zsq + wsq) - 2.0 * m
    dmin = jnp.min(dist, axis=1, keepdims=True)
    code_iota = jax.lax.broadcasted_iota(jnp.int32, (1024, NE), 1)
    idx = jnp.min(jnp.where(dist == dmin, code_iota, jnp.int32(NE)), axis=1)
    idx_ref[0, 0] = idx

    onehot = (code_iota == idx[:, None]).astype(jnp.float32)
    zq = jax.lax.dot_general(w, onehot, (((0,), (1,)), ((), ())),
                             preferred_element_type=jnp.float32,
                             precision=jax.lax.Precision.HIGHEST)
    zq_ref[0] = zq

    cnt_part = jnp.sum(onehot, axis=0)[None, :]
    loss_part = jnp.sum(dmin)

    lane = jax.lax.broadcasted_iota(jnp.int32, (1, 128), 1)

    @pl.when(t == 0)
    def _():
        cnt_ref[...] = cnt_part
        stats_ref[...] = jnp.where(lane == 0, loss_part, 0.0)

    @pl.when(t > 0)
    def _():
        cnt_ref[...] += cnt_part
        stats_ref[...] += jnp.where(lane == 0, loss_part, 0.0)

    @pl.when(t == NB - 1)
    def _():
        counts = cnt_ref[0, :]
        loss_sum = stats_ref[0, 0]
        probs = counts / jnp.float32(NPIX)
        safe = jnp.where(probs > 0, probs, 1.0)
        ent = -jnp.sum(jnp.where(probs > 0, probs * jnp.log(safe), 0.0))
        loss = loss_sum * (0.25 / (NPIX * ED))
        perp = jnp.exp(ent)
        usage = jnp.sum((counts > 0).astype(jnp.float32)) / jnp.float32(NE)
        stats_ref[...] = (jnp.where(lane == 0, loss, 0.0)
                          + jnp.where(lane == 1, perp, 0.0)
                          + jnp.where(lane == 2, usage, 0.0))


def _vq_call(z3, W, interpret=False):
    return pl.pallas_call(
        _vq_body,
        grid=(NB,),
        in_specs=[
            pl.BlockSpec((1, ED, 1024), lambda t: (t, 0, 0)),
            pl.BlockSpec((NE, ED), lambda t: (0, 0)),
        ],
        out_specs=[
            pl.BlockSpec((1, 1, 1024), lambda t: (t, 0, 0)),
            pl.BlockSpec((1, ED, 1024), lambda t: (t, 0, 0)),
            pl.BlockSpec((1, NE), lambda t: (0, 0)),
            pl.BlockSpec((1, 128), lambda t: (0, 0)),
        ],
        out_shape=[
            jax.ShapeDtypeStruct((NB, 1, 1024), jnp.int32),
            jax.ShapeDtypeStruct((NB, ED, 1024), jnp.float32),
            jax.ShapeDtypeStruct((1, NE), jnp.float32),
            jax.ShapeDtypeStruct((1, 128), jnp.float32),
        ],
        compiler_params=pltpu.CompilerParams(
            dimension_semantics=("arbitrary",),
        ),
        interpret=interpret,
    )(z3, W)


def kernel(z, W):
    B, D, H, Wd = z.shape
    z3 = z.reshape(B, D, H * Wd)
    idx3, zq3, _cnt, stats = _vq_call(z3, W)
    z_q = zq3.reshape(B, D, H, Wd)
    encoding_indices = idx3.reshape(-1)
    return (z_q, stats[0, 0], stats[0, 1], stats[0, 2], encoding_indices)


# speedup vs baseline: 1.9224x; 1.9224x over previous
import jax
import jax.numpy as jnp
from jax.experimental import pallas as pl
from jax.experimental.pallas import tpu as pltpu

NE = 1024
ED = 64
NPIX = 16384
TP = 2048
NT = NPIX // TP


def _vq_body(zf_ref, w_ref, idx_ref, zq_ref, cnt_ref, stats_ref):
    t = pl.program_id(0)
    zt = zf_ref[...]
    w = w_ref[...]
    m = jax.lax.dot_general(zt, w, (((1,), (1,)), ((), ())),
                            preferred_element_type=jnp.float32)
    zsq = jnp.sum(zt * zt, axis=1, keepdims=True)
    wsq = jnp.sum(w * w, axis=1)[None, :]
    dist = (zsq + wsq) - 2.0 * m
    dmin = jnp.min(dist, axis=1, keepdims=True)
    code_iota = jax.lax.broadcasted_iota(jnp.int32, (TP, NE), 1)
    idx = jnp.min(jnp.where(dist == dmin, code_iota, jnp.int32(NE)), axis=1)
    idx_ref[0, 0] = idx

    onehot = (code_iota == idx[:, None]).astype(jnp.float32)
    zq = jax.lax.dot_general(onehot.astype(jnp.bfloat16), w.astype(jnp.bfloat16),
                             (((1,), (0,)), ((), ())),
                             preferred_element_type=jnp.float32)
    zq_ref[...] = zq

    cnt_part = jnp.sum(onehot, axis=0)[None, :]
    loss_part = jnp.sum(dmin)

    lane = jax.lax.broadcasted_iota(jnp.int32, (1, 128), 1)

    @pl.when(t == 0)
    def _():
        cnt_ref[...] = cnt_part
        stats_ref[...] = jnp.where(lane == 0, loss_part, 0.0)

    @pl.when(t > 0)
    def _():
        cnt_ref[...] += cnt_part
        stats_ref[...] += jnp.where(lane == 0, loss_part, 0.0)

    @pl.when(t == NT - 1)
    def _():
        counts = cnt_ref[0, :]
        loss_sum = stats_ref[0, 0]
        probs = counts / jnp.float32(NPIX)
        safe = jnp.where(probs > 0, probs, 1.0)
        ent = -jnp.sum(jnp.where(probs > 0, probs * jnp.log(safe), 0.0))
        loss = loss_sum * (0.25 / (NPIX * ED))
        perp = jnp.exp(ent)
        usage = jnp.sum((counts > 0).astype(jnp.float32)) / jnp.float32(NE)
        stats_ref[...] = (jnp.where(lane == 0, loss, 0.0)
                          + jnp.where(lane == 1, perp, 0.0)
                          + jnp.where(lane == 2, usage, 0.0))


def _vq_call(z_flat, W, interpret=False):
    return pl.pallas_call(
        _vq_body,
        grid=(NT,),
        in_specs=[
            pl.BlockSpec((TP, ED), lambda t: (t, 0)),
            pl.BlockSpec((NE, ED), lambda t: (0, 0)),
        ],
        out_specs=[
            pl.BlockSpec((1, 1, TP), lambda t: (t, 0, 0)),
            pl.BlockSpec((TP, ED), lambda t: (t, 0)),
            pl.BlockSpec((1, NE), lambda t: (0, 0)),
            pl.BlockSpec((1, 128), lambda t: (0, 0)),
        ],
        out_shape=[
            jax.ShapeDtypeStruct((NT, 1, TP), jnp.int32),
            jax.ShapeDtypeStruct((NPIX, ED), jnp.float32),
            jax.ShapeDtypeStruct((1, NE), jnp.float32),
            jax.ShapeDtypeStruct((1, 128), jnp.float32),
        ],
        compiler_params=pltpu.CompilerParams(
            dimension_semantics=("arbitrary",),
        ),
        interpret=interpret,
    )(z_flat, W)


def kernel(z, W):
    B, D, H, Wd = z.shape
    z_flat = jnp.transpose(z, (0, 2, 3, 1)).reshape(-1, D)
    idx3, zq_flat, _cnt, stats = _vq_call(z_flat, W)
    z_q = jnp.transpose(zq_flat.reshape(B, H, Wd, D), (0, 3, 1, 2))
    encoding_indices = idx3.reshape(-1)
    return (z_q, stats[0, 0], stats[0, 1], stats[0, 2], encoding_indices)


# speedup vs baseline: 1.9661x; 1.0227x over previous
import jax
import jax.numpy as jnp
from jax.experimental import pallas as pl
from jax.experimental.pallas import tpu as pltpu

NE = 1024
ED = 64
NPIX = 16384
TP = 4096
NT = NPIX // TP


def _vq_body(zf_ref, w_ref, idx_ref, zq_ref, cnt_ref, stats_ref):
    t = pl.program_id(0)
    zt = zf_ref[...]
    w = w_ref[...]
    m = jax.lax.dot_general(zt, w, (((1,), (1,)), ((), ())),
                            preferred_element_type=jnp.float32)
    zsq = jnp.sum(zt * zt, axis=1, keepdims=True)
    wsq = jnp.sum(w * w, axis=1)[None, :]
    dist = (zsq + wsq) - 2.0 * m
    dmin = jnp.min(dist, axis=1, keepdims=True)
    code_iota = jax.lax.broadcasted_iota(jnp.int32, (TP, NE), 1)
    idx = jnp.min(jnp.where(dist == dmin, code_iota, jnp.int32(NE)), axis=1)
    idx_ref[0, 0] = idx

    onehot = (code_iota == idx[:, None]).astype(jnp.float32)
    zq = jax.lax.dot_general(onehot.astype(jnp.bfloat16), w.astype(jnp.bfloat16),
                             (((1,), (0,)), ((), ())),
                             preferred_element_type=jnp.float32)
    zq_ref[...] = zq

    cnt_part = jnp.sum(onehot, axis=0)[None, :]
    loss_part = jnp.sum(dmin)

    lane = jax.lax.broadcasted_iota(jnp.int32, (1, 128), 1)

    @pl.when(t == 0)
    def _():
        cnt_ref[...] = cnt_part
        stats_ref[...] = jnp.where(lane == 0, loss_part, 0.0)

    @pl.when(t > 0)
    def _():
        cnt_ref[...] += cnt_part
        stats_ref[...] += jnp.where(lane == 0, loss_part, 0.0)

    @pl.when(t == NT - 1)
    def _():
        counts = cnt_ref[0, :]
        loss_sum = stats_ref[0, 0]
        probs = counts / jnp.float32(NPIX)
        safe = jnp.where(probs > 0, probs, 1.0)
        ent = -jnp.sum(jnp.where(probs > 0, probs * jnp.log(safe), 0.0))
        loss = loss_sum * (0.25 / (NPIX * ED))
        perp = jnp.exp(ent)
        usage = jnp.sum((counts > 0).astype(jnp.float32)) / jnp.float32(NE)
        stats_ref[...] = (jnp.where(lane == 0, loss, 0.0)
                          + jnp.where(lane == 1, perp, 0.0)
                          + jnp.where(lane == 2, usage, 0.0))


def _vq_call(z_flat, W, interpret=False):
    return pl.pallas_call(
        _vq_body,
        grid=(NT,),
        in_specs=[
            pl.BlockSpec((TP, ED), lambda t: (t, 0)),
            pl.BlockSpec((NE, ED), lambda t: (0, 0)),
        ],
        out_specs=[
            pl.BlockSpec((1, 1, TP), lambda t: (t, 0, 0)),
            pl.BlockSpec((TP, ED), lambda t: (t, 0)),
            pl.BlockSpec((1, NE), lambda t: (0, 0)),
            pl.BlockSpec((1, 128), lambda t: (0, 0)),
        ],
        out_shape=[
            jax.ShapeDtypeStruct((NT, 1, TP), jnp.int32),
            jax.ShapeDtypeStruct((NPIX, ED), jnp.float32),
            jax.ShapeDtypeStruct((1, NE), jnp.float32),
            jax.ShapeDtypeStruct((1, 128), jnp.float32),
        ],
        compiler_params=pltpu.CompilerParams(
            dimension_semantics=("arbitrary",),
        ),
        interpret=interpret,
    )(z_flat, W)


def kernel(z, W):
    B, D, H, Wd = z.shape
    z_flat = jnp.transpose(z, (0, 2, 3, 1)).reshape(-1, D)
    idx3, zq_flat, _cnt, stats = _vq_call(z_flat, W)
    z_q = jnp.transpose(zq_flat.reshape(B, H, Wd, D), (0, 3, 1, 2))
    encoding_indices = idx3.reshape(-1)
    return (z_q, stats[0, 0], stats[0, 1], stats[0, 2], encoding_indices)
